# baseline (speedup 1.0000x reference)
import functools

import jax
import jax.numpy as jnp
from jax.experimental import pallas as pl
from jax.experimental.pallas import tpu as pltpu

B = 128
V = 100000
K = 10000
BM = 8

_TINY = 1.17549435e-38


def _sortable_key(x):
    i = x.view(jnp.int32)
    int_min = jnp.int32(-2147483648)
    return jnp.where(i < 0, int_min - i, i)


def _threefry_bits(flat_idx):
    k0 = jnp.int32(0)
    k1 = jnp.int32(42)
    k2 = jnp.int32(0x1BD11BDA) ^ k0 ^ k1
    ks = (k0, k1, k2)
    rot_a = (13, 15, 26, 6)
    rot_b = (17, 29, 16, 24)

    def rotl(v, d):
        return jax.lax.shift_left(v, jnp.int32(d)) | jax.lax.shift_right_logical(
            v, jnp.int32(32 - d)
        )

    x0 = jnp.full_like(flat_idx, k0)
    x1 = flat_idx + k1

    def four_rounds(x0, x1, rots):
        for r in rots:
            x0 = x0 + x1
            x1 = x0 ^ rotl(x1, r)
        return x0, x1

    for i in range(5):
        x0, x1 = four_rounds(x0, x1, rot_a if i % 2 == 0 else rot_b)
        x0 = x0 + ks[(i + 1) % 3]
        x1 = x1 + ks[(i + 2) % 3] + jnp.int32(i + 1)
    return x0 ^ x1


NFUSED = 20
CH = 5120


def _gumbel_chunk(block_row0, start):
    row = block_row0 + jax.lax.broadcasted_iota(jnp.int32, (BM, CH), 0)
    col = start + jax.lax.broadcasted_iota(jnp.int32, (BM, CH), 1)
    bits = _threefry_bits(row * V + col)
    fb = jax.lax.shift_right_logical(bits, jnp.int32(9)) | jnp.int32(0x3F800000)
    f = fb.view(jnp.float32) - jnp.float32(1.0)
    u = jnp.maximum(f, _TINY)
    return -jnp.log(-jnp.log(u))


def _body(x_ref, probs_ref, sample_ref, g_ref):
    x = x_ref[...]
    key = _sortable_key(x)

    lo0 = jnp.full((BM, 1), -0x7F800001, jnp.int32)
    hi0 = jnp.full((BM, 1), 0x7F800001, jnp.int32)

    s_lo = jnp.int32(0x3FA00000)
    s_hi = jnp.int32(0x3FB00000)
    cnt_lo = jnp.sum((key >= s_lo).astype(jnp.int32), axis=1, keepdims=True)
    cnt_hi = jnp.sum((key >= s_hi).astype(jnp.int32), axis=1, keepdims=True)
    lo0 = jnp.where(cnt_lo >= K, jnp.full((BM, 1), s_lo), lo0)
    hi0 = jnp.where(cnt_hi < K, jnp.full((BM, 1), s_hi), hi0)

    ones_v = jnp.full((V, 1), 1.0, jnp.float32)
    block_row0 = pl.program_id(0) * BM
    kf = jnp.float32(K)

    def mxu_count(mask):
        return jax.lax.dot_general(
            mask.astype(jnp.float32),
            ones_v,
            (((1,), (0,)), ((), ())),
            preferred_element_type=jnp.float32,
        )

    def fused_step(i, carry):
        lo, hi = carry
        mid = (lo >> 1) + (hi >> 1) + (lo & hi & 1)
        cnt = mxu_count(key >= mid)
        start = i * CH
        g_ref[:, pl.ds(start, CH)] = _gumbel_chunk(block_row0, start)
        ge = cnt >= kf
        return jnp.where(ge, mid, lo), jnp.where(ge, hi, mid)

    lo, hi = jax.lax.fori_loop(0, NFUSED, fused_step, (lo0, hi0))

    def bisect_cond(carry):
        lo, hi = carry
        d = hi - lo
        return jnp.any((d != 0) & (d != 1))

    def bisect_step(carry):
        lo, hi = carry
        mid = (lo >> 1) + (hi >> 1) + (lo & hi & 1)
        ge = mxu_count(key >= mid) >= kf
        return jnp.where(ge, mid, lo), jnp.where(ge, hi, mid)

    lo, _ = jax.lax.while_loop(bisect_cond, bisect_step, (lo, hi))
    keep = key >= lo

    m = jnp.max(x, axis=1, keepdims=True)
    e = jnp.where(keep, jnp.exp(x - m), jnp.float32(0.0))
    z = jnp.sum(e, axis=1, keepdims=True)
    probs_ref[...] = e / z

    col = jax.lax.broadcasted_iota(jnp.int32, (BM, V), 1)
    g = g_ref[:, :V]
    score = jnp.where(keep, x + g, -jnp.inf)
    best = jnp.max(score, axis=1, keepdims=True)
    idx = jnp.min(jnp.where(score == best, col, jnp.int32(V)), axis=1, keepdims=True)
    sample_ref[...] = idx


@jax.jit
def kernel(logits):
    probs, sample = pl.pallas_call(
        _body,
        grid=(B // BM,),
        in_specs=[pl.BlockSpec((BM, V), lambda i: (i, 0))],
        out_specs=[
            pl.BlockSpec((BM, V), lambda i: (i, 0)),
            pl.BlockSpec((BM, 1), lambda i: (i, 0)),
        ],
        out_shape=[
            jax.ShapeDtypeStruct((B, V), jnp.float32),
            jax.ShapeDtypeStruct((B, 1), jnp.int32),
        ],
        scratch_shapes=[pltpu.VMEM((BM, NFUSED * CH), jnp.float32)],
        compiler_params=pltpu.CompilerParams(
            dimension_semantics=("parallel",),
        ),
    )(logits)
    return probs, sample


# speedup vs baseline: 1.9642x; 1.9642x over previous
import functools

import jax
import jax.numpy as jnp
from jax.experimental import pallas as pl
from jax.experimental.pallas import tpu as pltpu

B = 128
V = 100000
K = 10000
BM = 8

_TINY = 1.17549435e-38


def _sortable_key(x):
    i = x.view(jnp.int32)
    int_min = jnp.int32(-2147483648)
    return jnp.where(i < 0, int_min - i, i)


def _threefry_bits(flat_idx):
    k0 = jnp.int32(0)
    k1 = jnp.int32(42)
    k2 = jnp.int32(0x1BD11BDA) ^ k0 ^ k1
    ks = (k0, k1, k2)
    rot_a = (13, 15, 26, 6)
    rot_b = (17, 29, 16, 24)

    def rotl(v, d):
        return jax.lax.shift_left(v, jnp.int32(d)) | jax.lax.shift_right_logical(
            v, jnp.int32(32 - d)
        )

    x0 = jnp.full_like(flat_idx, k0)
    x1 = flat_idx + k1

    def four_rounds(x0, x1, rots):
        for r in rots:
            x0 = x0 + x1
            x1 = x0 ^ rotl(x1, r)
        return x0, x1

    for i in range(5):
        x0, x1 = four_rounds(x0, x1, rot_a if i % 2 == 0 else rot_b)
        x0 = x0 + ks[(i + 1) % 3]
        x1 = x1 + ks[(i + 2) % 3] + jnp.int32(i + 1)
    return x0 ^ x1


NFUSED = 20
CH = 5120
NFULL = V // CH


def _body(x_ref, probs_ref, sample_ref):
    x = x_ref[...]
    key = _sortable_key(x)

    lo0 = jnp.full((BM, 1), -0x7F800001, jnp.int32)
    hi0 = jnp.full((BM, 1), 0x7F800001, jnp.int32)

    s_lo = jnp.int32(0x3FA00000)
    s_hi = jnp.int32(0x3FB00000)
    cnt_lo = jnp.sum((key >= s_lo).astype(jnp.int32), axis=1, keepdims=True)
    cnt_hi = jnp.sum((key >= s_hi).astype(jnp.int32), axis=1, keepdims=True)
    lo0 = jnp.where(cnt_lo >= K, jnp.full((BM, 1), s_lo), lo0)
    hi0 = jnp.where(cnt_hi < K, jnp.full((BM, 1), s_hi), hi0)

    block_row0 = pl.program_id(0) * BM

    def fused_step(i, carry):
        lo, hi = carry
        mid = (lo >> 1) + (hi >> 1) + (lo & hi & 1)
        cnt = jnp.sum((key >= mid).astype(jnp.int32), axis=1, keepdims=True)
        ge = cnt >= K
        return jnp.where(ge, mid, lo), jnp.where(ge, hi, mid)

    lo, hi = jax.lax.fori_loop(0, NFUSED, fused_step, (lo0, hi0))

    def bisect_cond(carry):
        lo, hi = carry
        d = hi - lo
        return jnp.any((d != 0) & (d != 1))

    def bisect_step(carry):
        lo, hi = carry
        mid = (lo >> 1) + (hi >> 1) + (lo & hi & 1)
        cnt = jnp.sum((key >= mid).astype(jnp.int32), axis=1, keepdims=True)
        ge = cnt >= K
        return jnp.where(ge, mid, lo), jnp.where(ge, hi, mid)

    lo, _ = jax.lax.while_loop(bisect_cond, bisect_step, (lo, hi))
    keep = key >= lo

    m = jnp.max(x, axis=1, keepdims=True)
    e = jnp.where(keep, jnp.exp(x - m), jnp.float32(0.0))
    z = jnp.sum(e, axis=1, keepdims=True)
    probs_ref[...] = e / z

    def chunk_update(start, width, best, bidx):
        xc = x_ref[:, pl.ds(start, width)]
        kc = _sortable_key(xc) >= lo
        row = block_row0 + jax.lax.broadcasted_iota(jnp.int32, (BM, width), 0)
        col = start + jax.lax.broadcasted_iota(jnp.int32, (BM, width), 1)
        bits = _threefry_bits(row * V + col)
        fb = jax.lax.shift_right_logical(bits, jnp.int32(9)) | jnp.int32(0x3F800000)
        u = jnp.maximum(fb.view(jnp.float32) - jnp.float32(1.0), _TINY)
        g = -jnp.log(-jnp.log(u))
        score = jnp.where(kc, xc + g, -jnp.inf)
        cmax = jnp.max(score, axis=1, keepdims=True)
        cidx = jnp.min(
            jnp.where(score == cmax, col, jnp.int32(V)), axis=1, keepdims=True
        )
        upd = cmax > best
        return jnp.where(upd, cmax, best), jnp.where(upd, cidx, bidx)

    best0 = jnp.full((BM, 1), -jnp.inf, jnp.float32)
    bidx0 = jnp.full((BM, 1), V, jnp.int32)

    def stream_step(i, carry):
        return chunk_update(i * CH, CH, *carry)

    best, bidx = jax.lax.fori_loop(0, NFULL, stream_step, (best0, bidx0))
    _, bidx = chunk_update(NFULL * CH, V - NFULL * CH, best, bidx)
    sample_ref[...] = bidx


@jax.jit
def kernel(logits):
    probs, sample = pl.pallas_call(
        _body,
        grid=(B // BM,),
        in_specs=[pl.BlockSpec((BM, V), lambda i: (i, 0))],
        out_specs=[
            pl.BlockSpec((BM, V), lambda i: (i, 0)),
            pl.BlockSpec((BM, 1), lambda i: (i, 0)),
        ],
        out_shape=[
            jax.ShapeDtypeStruct((B, V), jnp.float32),
            jax.ShapeDtypeStruct((B, 1), jnp.int32),
        ],
        compiler_params=pltpu.CompilerParams(
            dimension_semantics=("parallel",),
        ),
    )(logits)
    return probs, sample


# speedup vs baseline: 2.4756x; 1.2603x over previous
import functools

import jax
import jax.numpy as jnp
from jax.experimental import pallas as pl
from jax.experimental.pallas import tpu as pltpu

B = 128
V = 100000
K = 10000
BM = 16

_TINY = 1.17549435e-38


def _sortable_key(x):
    i = x.view(jnp.int32)
    int_min = jnp.int32(-2147483648)
    return jnp.where(i < 0, int_min - i, i)


def _threefry_bits(flat_idx):
    k0 = jnp.int32(0)
    k1 = jnp.int32(42)
    k2 = jnp.int32(0x1BD11BDA) ^ k0 ^ k1
    ks = (k0, k1, k2)
    rot_a = (13, 15, 26, 6)
    rot_b = (17, 29, 16, 24)

    def rotl(v, d):
        return jax.lax.shift_left(v, jnp.int32(d)) | jax.lax.shift_right_logical(
            v, jnp.int32(32 - d)
        )

    x0 = jnp.full_like(flat_idx, k0)
    x1 = flat_idx + k1

    def four_rounds(x0, x1, rots):
        for r in rots:
            x0 = x0 + x1
            x1 = x0 ^ rotl(x1, r)
        return x0, x1

    for i in range(5):
        x0, x1 = four_rounds(x0, x1, rot_a if i % 2 == 0 else rot_b)
        x0 = x0 + ks[(i + 1) % 3]
        x1 = x1 + ks[(i + 2) % 3] + jnp.int32(i + 1)
    return x0 ^ x1


NFUSED = 20
CH = 5120


def _gumbel_chunk(block_row0, start):
    row = block_row0 + jax.lax.broadcasted_iota(jnp.int32, (BM, CH), 0)
    col = start + jax.lax.broadcasted_iota(jnp.int32, (BM, CH), 1)
    bits = _threefry_bits(row * V + col)
    fb = jax.lax.shift_right_logical(bits, jnp.int32(9)) | jnp.int32(0x3F800000)
    f = fb.view(jnp.float32) - jnp.float32(1.0)
    u = jnp.maximum(f, _TINY)
    return -jnp.log(-jnp.log(u))


def _body(x_ref, probs_ref, sample_ref, g_ref):
    x = x_ref[...]
    key = _sortable_key(x)

    lo0 = jnp.full((BM, 1), -0x7F800001, jnp.int32)
    hi0 = jnp.full((BM, 1), 0x7F800001, jnp.int32)

    s_lo = jnp.int32(0x3FA00000)
    s_hi = jnp.int32(0x3FB00000)
    cnt_lo = jnp.sum((key >= s_lo).astype(jnp.int32), axis=1, keepdims=True)
    cnt_hi = jnp.sum((key >= s_hi).astype(jnp.int32), axis=1, keepdims=True)
    lo0 = jnp.where(cnt_lo >= K, jnp.full((BM, 1), s_lo), lo0)
    hi0 = jnp.where(cnt_hi < K, jnp.full((BM, 1), s_hi), hi0)

    block_row0 = pl.program_id(0) * BM

    def fused_step(i, carry):
        lo, hi = carry
        mid = (lo >> 1) + (hi >> 1) + (lo & hi & 1)
        cnt = jnp.sum((key >= mid).astype(jnp.int32), axis=1, keepdims=True)
        start = i * CH
        g_ref[:, pl.ds(start, CH)] = _gumbel_chunk(block_row0, start)
        ge = cnt >= K
        return jnp.where(ge, mid, lo), jnp.where(ge, hi, mid)

    lo, hi = jax.lax.fori_loop(0, NFUSED, fused_step, (lo0, hi0))

    def bisect_cond(carry):
        lo, hi = carry
        d = hi - lo
        return jnp.any((d != 0) & (d != 1))

    def bisect_step(carry):
        lo, hi = carry
        mid = (lo >> 1) + (hi >> 1) + (lo & hi & 1)
        cnt = jnp.sum((key >= mid).astype(jnp.int32), axis=1, keepdims=True)
        ge = cnt >= K
        return jnp.where(ge, mid, lo), jnp.where(ge, hi, mid)

    lo, _ = jax.lax.while_loop(bisect_cond, bisect_step, (lo, hi))
    keep = key >= lo

    m = jnp.max(x, axis=1, keepdims=True)
    e = jnp.where(keep, jnp.exp(x - m), jnp.float32(0.0))
    z = jnp.sum(e, axis=1, keepdims=True)
    probs_ref[...] = e / z

    col = jax.lax.broadcasted_iota(jnp.int32, (BM, V), 1)
    g = g_ref[:, :V]
    score = jnp.where(keep, x + g, -jnp.inf)
    best = jnp.max(score, axis=1, keepdims=True)
    idx = jnp.min(jnp.where(score == best, col, jnp.int32(V)), axis=1, keepdims=True)
    sample_ref[...] = idx


@jax.jit
def kernel(logits):
    probs, sample = pl.pallas_call(
        _body,
        grid=(B // BM,),
        in_specs=[pl.BlockSpec((BM, V), lambda i: (i, 0))],
        out_specs=[
            pl.BlockSpec((BM, V), lambda i: (i, 0)),
            pl.BlockSpec((BM, 1), lambda i: (i, 0)),
        ],
        out_shape=[
            jax.ShapeDtypeStruct((B, V), jnp.float32),
            jax.ShapeDtypeStruct((B, 1), jnp.int32),
        ],
        scratch_shapes=[pltpu.VMEM((BM, NFUSED * CH), jnp.float32)],
        compiler_params=pltpu.CompilerParams(
            dimension_semantics=("parallel",),
        ),
    )(logits)
    return probs, sample


# speedup vs baseline: 2.4766x; 1.0004x over previous
import functools

import jax
import jax.numpy as jnp
from jax.experimental import pallas as pl
from jax.experimental.pallas import tpu as pltpu

B = 128
V = 100000
K = 10000
BM = 16

_TINY = 1.17549435e-38


def _sortable_key(x):
    i = x.view(jnp.int32)
    int_min = jnp.int32(-2147483648)
    return jnp.where(i < 0, int_min - i, i)


def _threefry_bits(flat_idx):
    k0 = jnp.int32(0)
    k1 = jnp.int32(42)
    k2 = jnp.int32(0x1BD11BDA) ^ k0 ^ k1
    ks = (k0, k1, k2)
    rot_a = (13, 15, 26, 6)
    rot_b = (17, 29, 16, 24)

    def rotl(v, d):
        return jax.lax.shift_left(v, jnp.int32(d)) | jax.lax.shift_right_logical(
            v, jnp.int32(32 - d)
        )

    x0 = jnp.full_like(flat_idx, k0)
    x1 = flat_idx + k1

    def four_rounds(x0, x1, rots):
        for r in rots:
            x0 = x0 + x1
            x1 = x0 ^ rotl(x1, r)
        return x0, x1

    for i in range(5):
        x0, x1 = four_rounds(x0, x1, rot_a if i % 2 == 0 else rot_b)
        x0 = x0 + ks[(i + 1) % 3]
        x1 = x1 + ks[(i + 2) % 3] + jnp.int32(i + 1)
    return x0 ^ x1


NFUSED = 20
CH = 5120


def _gumbel_chunk(block_row0, start):
    row = block_row0 + jax.lax.broadcasted_iota(jnp.int32, (BM, CH), 0)
    col = start + jax.lax.broadcasted_iota(jnp.int32, (BM, CH), 1)
    bits = _threefry_bits(row * V + col)
    fb = jax.lax.shift_right_logical(bits, jnp.int32(9)) | jnp.int32(0x3F800000)
    f = fb.view(jnp.float32) - jnp.float32(1.0)
    u = jnp.maximum(f, _TINY)
    return -jnp.log(-jnp.log(u))


def _body(x_ref, probs_ref, sample_ref, g_ref):
    x = x_ref[...]
    key = _sortable_key(x)

    lo0 = jnp.full((BM, 1), -0x7F800001, jnp.int32)
    hi0 = jnp.full((BM, 1), 0x7F800001, jnp.int32)

    s_lo = jnp.int32(0x3FA00000)
    s_hi = jnp.int32(0x3FB00000)
    cnt_lo = jnp.sum((key >= s_lo).astype(jnp.int32), axis=1, keepdims=True)
    cnt_hi = jnp.sum((key >= s_hi).astype(jnp.int32), axis=1, keepdims=True)
    lo0 = jnp.where(cnt_lo >= K, jnp.full((BM, 1), s_lo), lo0)
    hi0 = jnp.where(cnt_hi < K, jnp.full((BM, 1), s_hi), hi0)

    block_row0 = pl.program_id(0) * BM

    def fused_step(i, carry):
        lo, hi = carry
        mid = (lo >> 1) + (hi >> 1) + (lo & hi & 1)
        cnt = jnp.sum((key >= mid).astype(jnp.int32), axis=1, keepdims=True)
        start = i * CH
        g_ref[:, pl.ds(start, CH)] = _gumbel_chunk(block_row0, start)
        ge = cnt >= K
        return jnp.where(ge, mid, lo), jnp.where(ge, hi, mid)

    lo, hi = jax.lax.fori_loop(0, NFUSED, fused_step, (lo0, hi0))

    def bisect_cond(carry):
        lo, hi = carry
        d = hi - lo
        return jnp.any((d != 0) & (d != 1))

    def bisect_step(carry):
        lo, hi = carry
        mid = (lo >> 1) + (hi >> 1) + (lo & hi & 1)
        cnt = jnp.sum((key >= mid).astype(jnp.int32), axis=1, keepdims=True)
        ge = cnt >= K
        return jnp.where(ge, mid, lo), jnp.where(ge, hi, mid)

    lo, _ = jax.lax.while_loop(bisect_cond, bisect_step, (lo, hi))
    keep = key >= lo

    m = jnp.max(x, axis=1, keepdims=True)
    e = jnp.where(keep, jnp.exp(x - m), jnp.float32(0.0))
    z = jnp.sum(e, axis=1, keepdims=True)
    probs_ref[...] = e * (jnp.float32(1.0) / z)

    col = jax.lax.broadcasted_iota(jnp.int32, (BM, V), 1)
    g = g_ref[:, :V]
    score = jnp.where(keep, x + g, -jnp.inf)
    best = jnp.max(score, axis=1, keepdims=True)
    idx = jnp.min(jnp.where(score == best, col, jnp.int32(V)), axis=1, keepdims=True)
    sample_ref[...] = idx


@jax.jit
def kernel(logits):
    probs, sample = pl.pallas_call(
        _body,
        grid=(B // BM,),
        in_specs=[pl.BlockSpec((BM, V), lambda i: (i, 0))],
        out_specs=[
            pl.BlockSpec((BM, V), lambda i: (i, 0)),
            pl.BlockSpec((BM, 1), lambda i: (i, 0)),
        ],
        out_shape=[
            jax.ShapeDtypeStruct((B, V), jnp.float32),
            jax.ShapeDtypeStruct((B, 1), jnp.int32),
        ],
        scratch_shapes=[pltpu.VMEM((BM, NFUSED * CH), jnp.float32)],
        compiler_params=pltpu.CompilerParams(
            dimension_semantics=("parallel",),
        ),
    )(logits)
    return probs, sample
